# baseline (speedup 1.0000x reference)
import functools

import numpy as np
import jax
import jax.numpy as jnp
from jax import lax
from jax.experimental import pallas as pl
from jax.experimental.pallas import tpu as pltpu
from jax.experimental.pallas import tpu_sc as plsc

_PL, _PS, _LW = 16, 4, 4
_B, _L = 64, 4096
_P = (_L - _PL) // _PS + 1
_NG = _B * _P
_E0 = 108
_EINTRA = _NG * _E0
_CROSS = _B * (_P - 1)
_ET = _EINTRA + _CROSS
_NN = _NG * _PL

_NW = 32

_CL = 6912
_NCHUNK = _EINTRA // _CL
_FULL_T = _NCHUNK // _NW
_EXTRA = _NCHUNK - _FULL_T * _NW

_BA_CHUNK = 8192
_BA_SLAB = _NN // _NW
_BA_FULL = _BA_SLAB // _BA_CHUNK
_BA_TAIL = _BA_SLAB - _BA_FULL * _BA_CHUNK


def _base_tables():
    edges, dists = [], []
    for i in range(_PL):
        for j in range(_PL):
            d = abs(i - j)
            if 0 < d <= _LW:
                edges.append((i, j))
                dists.append(d - 1)
    e = np.array(edges, np.int32).T
    return e[0], e[1], np.array(dists, np.int32)


_BI, _BJ, _BD = _base_tables()
_U = np.arange(4 * _E0)
_PAT4_NP = np.stack([
    16 * (_U // _E0) + _BI[_U % _E0],
    16 * (_U // _E0) + _BJ[_U % _E0],
]).astype(np.int32)
_DIST4_NP = _BD[_U % _E0].astype(np.int32)

_MESH = plsc.VectorSubcoreMesh(core_axis_name="c", subcore_axis_name="s")


@functools.partial(
    pl.kernel,
    mesh=_MESH,
    out_type=[
        jax.ShapeDtypeStruct((2, _ET), jnp.int32),
        jax.ShapeDtypeStruct((_ET,), jnp.float32),
        jax.ShapeDtypeStruct((_NN,), jnp.int32),
    ],
    scratch_types=[
        pltpu.VMEM((2, _CL), jnp.int32),
        pltpu.VMEM((_CL,), jnp.float32),
        pltpu.VMEM((_BA_CHUNK,), jnp.int32),
        pltpu.VMEM((2, 2048), jnp.int32),
        pltpu.VMEM((2048,), jnp.float32),
        pltpu.VMEM((16,), jnp.float32),
        pltpu.VMEM((2, 4 * _E0), jnp.int32),
        pltpu.VMEM((4 * _E0,), jnp.int32),
    ],
)
def _sc_build(wpad, pat, dist,
              ei, attr, batch,
              ei_stg, at_stg, ba_stg, cei_stg, cat_stg,
              wv, patv, distv):
    w = lax.axis_index("s") * 2 + lax.axis_index("c")
    iota = lax.iota(jnp.int32, 16)

    pltpu.sync_copy(wpad, wv)
    pltpu.sync_copy(pat, patv)
    pltpu.sync_copy(dist, distv)
    wvec = wv[...]

    for v in range(27):
        dvec = distv[pl.ds(16 * v, 16)]
        at_stg[pl.ds(16 * v, 16)] = wvec.at[dvec].get(
            mode="promise_in_bounds")
    sz = 432
    while sz < _CL:
        n = min(sz, _CL - sz)

        def _dup(i, _, sz=sz):
            at_stg[pl.ds(sz + 16 * i, 16)] = at_stg[pl.ds(16 * i, 16)]
            return 0

        lax.fori_loop(0, n // 16, _dup, 0)
        sz += n

    def _at_dma(t, _):
        cid = w + _NW * t
        pltpu.sync_copy(at_stg, attr.at[pl.ds(_CL * cid, _CL)])
        return 0

    lax.fori_loop(0, _FULL_T, _at_dma, 0)

    @pl.when(w < _EXTRA)
    def _():
        _at_dma(_FULL_T, 0)

    cw = wvec.at[jnp.full((16,), 4, jnp.int32)].get(
        mode="promise_in_bounds")

    def _cat_fill(v, _):
        cat_stg[pl.ds(16 * v, 16)] = cw
        return 0

    lax.fori_loop(0, 128, _cat_fill, 0)

    cb1 = 2040 * w + 1020
    cb2 = 2040 * w + 2040

    def _cb_fill(v, _):
        m = jnp.full((16,), 2048 * w + 16 * v, jnp.int32) + iota
        step1 = jnp.minimum(jnp.maximum(m - (cb1 - 1), 0), 1)
        step2 = jnp.minimum(jnp.maximum(m - (cb2 - 1), 0), 1)
        s = 16 * m + (32 * w + 15) + 16 * step1 + 16 * step2
        cei_stg[0, pl.ds(16 * v, 16)] = s
        cei_stg[1, pl.ds(16 * v, 16)] = s + 1
        return 0

    lax.fori_loop(0, 128, _cb_fill, 0)

    @pl.when(w < _NW - 1)
    def _():
        pltpu.sync_copy(cei_stg,
                        ei.at[:, pl.ds(_EINTRA + 2048 * w, 2048)])
        pltpu.sync_copy(cat_stg,
                        attr.at[pl.ds(_EINTRA + 2048 * w, 2048)])

    @pl.when(w == _NW - 1)
    def _():
        pltpu.sync_copy(
            cei_stg.at[:, pl.ds(0, _CROSS - 2048 * (_NW - 1))],
            ei.at[:, pl.ds(_EINTRA + 2048 * (_NW - 1),
                           _CROSS - 2048 * (_NW - 1))])
        pltpu.sync_copy(
            cat_stg.at[pl.ds(0, _CROSS - 2048 * (_NW - 1))],
            attr.at[pl.ds(_EINTRA + 2048 * (_NW - 1),
                          _CROSS - 2048 * (_NW - 1))])

    for c in range(_BA_FULL + 1):
        nvec = (_BA_CHUNK if c < _BA_FULL else _BA_TAIL) // 16
        g0 = (_BA_SLAB // 16) * w + (_BA_CHUNK // 16) * c

        def _baf(i, _, g0=g0):
            ba_stg[pl.ds(16 * i, 16)] = jnp.full((16,), g0 + i, jnp.int32)
            return 0

        lax.fori_loop(0, nvec, _baf, 0)
        pltpu.sync_copy(
            ba_stg.at[pl.ds(0, nvec * 16)],
            batch.at[pl.ds(_BA_SLAB * w + _BA_CHUNK * c, nvec * 16)])

    pv = [[patv[r, pl.ds(16 * v, 16)] for v in range(27)] for r in range(2)]

    def _fill(q, g0):
        s = jnp.full((16,), 16 * (g0 + 4 * q), jnp.int32)
        for r in range(2):
            for v in range(27):
                ei_stg[r, pl.ds(432 * q + 16 * v, 16)] = pv[r][v] + s
        return g0

    def _eic(t, _):
        cid = w + _NW * t
        lax.fori_loop(0, 16, _fill, 64 * cid)
        pltpu.sync_copy(ei_stg, ei.at[:, pl.ds(_CL * cid, _CL)])
        return 0

    lax.fori_loop(0, _FULL_T, _eic, 0)

    @pl.when(w < _EXTRA)
    def _():
        _eic(_FULL_T, 0)


_NFS = 640
_NFB = 2048
_M_NP = np.zeros((_NFB, _NFS), np.float32)
_NARANGE = np.arange(_NFB)
_M_NP[_NARANGE, 4 * (_NARANGE // 16) + (_NARANGE % 16)] = 1.0


def _nf_body(x_ref, m_ref, o_ref):
    j = pl.program_id(1)
    xs = x_ref[0, :, pl.ds(512 * j, _NFS)]
    o_ref[0] = lax.dot_general(
        m_ref[...], xs, (((1,), (1,)), ((), ())),
        precision=lax.Precision.HIGHEST,
        preferred_element_type=jnp.float32)


def _nf_tc(xp, m):
    return pl.pallas_call(
        _nf_body,
        grid=(_B, 8),
        in_specs=[
            pl.BlockSpec((1, 2, _L + 128), lambda b, j: (b, 0, 0)),
            pl.BlockSpec((_NFB, _NFS), lambda b, j: (0, 0)),
        ],
        out_specs=pl.BlockSpec((1, _NFB, 2), lambda b, j: (b, j, 0)),
        out_shape=jax.ShapeDtypeStruct((_B, _P * _PL, 2), jnp.float32),
    )(xp, m)


def kernel(iq_signal, edge_weights, cross_patch_weight):
    wpad = (jnp.zeros((16,), jnp.float32)
            .at[:4].set(edge_weights)
            .at[4].set(cross_patch_weight))
    ei, attr, batch = _sc_build(
        wpad, jnp.asarray(_PAT4_NP), jnp.asarray(_DIST4_NP))
    xp = jnp.pad(iq_signal, ((0, 0), (0, 0), (0, 128)))
    nf = _nf_tc(xp, jnp.asarray(_M_NP)).reshape(_NN, 2)
    return nf, ei, attr, batch


# speedup vs baseline: 2.4011x; 2.4011x over previous
import functools

import numpy as np
import jax
import jax.numpy as jnp
from jax import lax
from jax.experimental import pallas as pl
from jax.experimental.pallas import tpu as pltpu
from jax.experimental.pallas import tpu_sc as plsc

_PL, _PS, _LW = 16, 4, 4
_B, _L = 64, 4096
_P = (_L - _PL) // _PS + 1
_NG = _B * _P
_E0 = 108
_EINTRA = _NG * _E0
_CROSS = _B * (_P - 1)
_ET = _EINTRA + _CROSS
_NN = _NG * _PL

_NW = 32

_CL = 6912
_NCHUNK = _EINTRA // _CL
_FULL_T = _NCHUNK // _NW
_EXTRA = _NCHUNK - _FULL_T * _NW

_BA_CHUNK = 8192
_BA_SLAB = _NN // _NW
_BA_FULL = _BA_SLAB // _BA_CHUNK
_BA_TAIL = _BA_SLAB - _BA_FULL * _BA_CHUNK


def _base_tables():
    edges, dists = [], []
    for i in range(_PL):
        for j in range(_PL):
            d = abs(i - j)
            if 0 < d <= _LW:
                edges.append((i, j))
                dists.append(d - 1)
    e = np.array(edges, np.int32).T
    return e[0], e[1], np.array(dists, np.int32)


_BI, _BJ, _BD = _base_tables()
_U = np.arange(4 * _E0)
_PAT4_NP = np.stack([
    16 * (_U // _E0) + _BI[_U % _E0],
    16 * (_U // _E0) + _BJ[_U % _E0],
]).astype(np.int32)
_DIST4_NP = _BD[_U % _E0].astype(np.int32)

_MESH = plsc.VectorSubcoreMesh(core_axis_name="c", subcore_axis_name="s")


@functools.partial(
    pl.kernel,
    mesh=_MESH,
    out_type=[
        jax.ShapeDtypeStruct((2 * _NN,), jnp.float32),
        jax.ShapeDtypeStruct((2, _ET), jnp.int32),
        jax.ShapeDtypeStruct((_ET,), jnp.float32),
        jax.ShapeDtypeStruct((_NN,), jnp.int32),
    ],
    scratch_types=[
        pltpu.VMEM((_P * _PL,), jnp.int32),
        pltpu.VMEM((_P * _PL,), jnp.int32),
        pltpu.VMEM((_P * _PL,), jnp.float32),
        pltpu.VMEM((_P * _PL,), jnp.float32),
        pltpu.VMEM((2, _CL), jnp.int32),
        pltpu.VMEM((_CL,), jnp.float32),
        pltpu.VMEM((_BA_CHUNK,), jnp.int32),
        pltpu.VMEM((2, 2048), jnp.int32),
        pltpu.VMEM((2048,), jnp.float32),
        pltpu.VMEM((16,), jnp.float32),
        pltpu.VMEM((2, 4 * _E0), jnp.int32),
        pltpu.VMEM((4 * _E0,), jnp.int32),
        pltpu.SemaphoreType.DMA,
        pltpu.SemaphoreType.DMA,
        pltpu.SemaphoreType.DMA,
    ],
)
def _sc_build(iq, wpad, pat, dist,
              nf, ei, attr, batch,
              idxa, idxb, stga, stgb, ei_stg, at_stg, ba_stg,
              cei_stg, cat_stg, wv, patv, distv,
              at_sem, sema, semb):
    w = lax.axis_index("s") * 2 + lax.axis_index("c")
    iota = lax.iota(jnp.int32, 16)

    pltpu.sync_copy(wpad, wv)
    pltpu.sync_copy(pat, patv)
    pltpu.sync_copy(dist, distv)
    wvec = wv[...]

    for v in range(27):
        dvec = distv[pl.ds(16 * v, 16)]
        at_stg[pl.ds(16 * v, 16)] = wvec.at[dvec].get(
            mode="promise_in_bounds")
    sz = 432
    while sz < _CL:
        n = min(sz, _CL - sz)

        def _dup(i, _, sz=sz):
            at_stg[pl.ds(sz + 16 * i, 16)] = at_stg[pl.ds(16 * i, 16)]
            return 0

        lax.fori_loop(0, n // 16, _dup, 0)
        sz += n

    def _at_dma(t, _):
        cid = w + _NW * t
        pltpu.async_copy(at_stg, attr.at[pl.ds(_CL * cid, _CL)], at_sem)
        return 0

    lax.fori_loop(0, _FULL_T, _at_dma, 0)

    @pl.when(w < _EXTRA)
    def _():
        _at_dma(_FULL_T, 0)

    cw = wvec.at[jnp.full((16,), 4, jnp.int32)].get(
        mode="promise_in_bounds")

    def _cat_fill(v, _):
        cat_stg[pl.ds(16 * v, 16)] = cw
        return 0

    lax.fori_loop(0, 128, _cat_fill, 0)

    cb1 = 2040 * w + 1020
    cb2 = 2040 * w + 2040

    def _cb_fill(v, _):
        m = jnp.full((16,), 2048 * w + 16 * v, jnp.int32) + iota
        step1 = jnp.minimum(jnp.maximum(m - (cb1 - 1), 0), 1)
        step2 = jnp.minimum(jnp.maximum(m - (cb2 - 1), 0), 1)
        s = 16 * m + (32 * w + 15) + 16 * step1 + 16 * step2
        cei_stg[0, pl.ds(16 * v, 16)] = s
        cei_stg[1, pl.ds(16 * v, 16)] = s + 1
        return 0

    lax.fori_loop(0, 128, _cb_fill, 0)

    @pl.when(w < _NW - 1)
    def _():
        pltpu.sync_copy(cei_stg,
                        ei.at[:, pl.ds(_EINTRA + 2048 * w, 2048)])
        pltpu.sync_copy(cat_stg,
                        attr.at[pl.ds(_EINTRA + 2048 * w, 2048)])

    @pl.when(w == _NW - 1)
    def _():
        pltpu.sync_copy(
            cei_stg.at[:, pl.ds(0, _CROSS - 2048 * (_NW - 1))],
            ei.at[:, pl.ds(_EINTRA + 2048 * (_NW - 1),
                           _CROSS - 2048 * (_NW - 1))])
        pltpu.sync_copy(
            cat_stg.at[pl.ds(0, _CROSS - 2048 * (_NW - 1))],
            attr.at[pl.ds(_EINTRA + 2048 * (_NW - 1),
                          _CROSS - 2048 * (_NW - 1))])

    for c in range(_BA_FULL + 1):
        nvec = (_BA_CHUNK if c < _BA_FULL else _BA_TAIL) // 16
        g0 = (_BA_SLAB // 16) * w + (_BA_CHUNK // 16) * c

        def _baf(i, _, g0=g0):
            ba_stg[pl.ds(16 * i, 16)] = jnp.full((16,), g0 + i, jnp.int32)
            return 0

        lax.fori_loop(0, nvec, _baf, 0)
        pltpu.sync_copy(
            ba_stg.at[pl.ds(0, nvec * 16)],
            batch.at[pl.ds(_BA_SLAB * w + _BA_CHUNK * c, nvec * 16)])

    _H = _P * _PL
    _NFV = _H // 16

    def _mkidx(ref, off):
        def _idx(v, _):
            j = jnp.full((16,), off + 16 * v, jnp.int32) + iota
            n = lax.shift_right_logical(j, 1)
            c = lax.bitwise_and(j, 1)
            p = lax.shift_right_logical(n, 4)
            k = lax.bitwise_and(n, 15)
            ref[pl.ds(16 * v, 16)] = (
                jnp.full((16,), 8192 * 2 * w, jnp.int32)
                + 4096 * c + 4 * p + k)
            return 0

        lax.fori_loop(0, _NFV, _idx, 0)

    def _bumpref(ref):
        def _b(v, _):
            ref[pl.ds(16 * v, 16)] = ref[pl.ds(16 * v, 16)] + 8192
            return 0

        lax.fori_loop(0, _NFV, _b, 0)

    _mkidx(idxa, 0)
    _mkidx(idxb, _H)

    pltpu.sync_copy(iq.at[idxa], stga)
    ha = pltpu.async_copy(stga, nf.at[pl.ds(2 * _H * 2 * w, _H)], sema)
    pltpu.sync_copy(iq.at[idxb], stgb)
    hb = pltpu.async_copy(stgb, nf.at[pl.ds(2 * _H * 2 * w + _H, _H)], semb)
    _bumpref(idxa)
    _bumpref(idxb)
    ha.wait()
    pltpu.sync_copy(iq.at[idxa], stga)
    ha2 = pltpu.async_copy(stga, nf.at[pl.ds(2 * _H * (2 * w + 1), _H)],
                           sema)
    hb.wait()
    pltpu.sync_copy(iq.at[idxb], stgb)
    hb2 = pltpu.async_copy(stgb, nf.at[pl.ds(2 * _H * (2 * w + 1) + _H, _H)],
                           semb)

    pv = [[patv[r, pl.ds(16 * v, 16)] for v in range(27)] for r in range(2)]

    def _fill(q, g0):
        s = jnp.full((16,), 16 * (g0 + 4 * q), jnp.int32)
        for r in range(2):
            for v in range(27):
                ei_stg[r, pl.ds(432 * q + 16 * v, 16)] = pv[r][v] + s
        return g0

    def _eic(t, _):
        cid = w + _NW * t
        lax.fori_loop(0, 16, _fill, 64 * cid)
        pltpu.sync_copy(ei_stg, ei.at[:, pl.ds(_CL * cid, _CL)])
        return 0

    lax.fori_loop(0, _FULL_T, _eic, 0)

    @pl.when(w < _EXTRA)
    def _():
        _eic(_FULL_T, 0)

    ha2.wait()
    hb2.wait()

    def _at_drain(t, _):
        pltpu.make_async_copy(
            attr.at[pl.ds(0, _CL)], at_stg, at_sem).wait()
        return 0

    lax.fori_loop(0, _FULL_T, _at_drain, 0)

    @pl.when(w < _EXTRA)
    def _():
        _at_drain(_FULL_T, 0)


def kernel(iq_signal, edge_weights, cross_patch_weight):
    wpad = (jnp.zeros((16,), jnp.float32)
            .at[:4].set(edge_weights)
            .at[4].set(cross_patch_weight))
    nf_flat, ei, attr, batch = _sc_build(
        iq_signal.reshape(-1), wpad,
        jnp.asarray(_PAT4_NP), jnp.asarray(_DIST4_NP))
    return nf_flat.reshape(_NN, 2), ei, attr, batch


# speedup vs baseline: 2.5437x; 1.0594x over previous
import functools

import numpy as np
import jax
import jax.numpy as jnp
from jax import lax
from jax.experimental import pallas as pl
from jax.experimental.pallas import tpu as pltpu
from jax.experimental.pallas import tpu_sc as plsc

_PL, _PS, _LW = 16, 4, 4
_B, _L = 64, 4096
_P = (_L - _PL) // _PS + 1
_NG = _B * _P
_E0 = 108
_EINTRA = _NG * _E0
_CROSS = _B * (_P - 1)
_ET = _EINTRA + _CROSS
_NN = _NG * _PL

_NW = 32

_CL = 6912
_NCHUNK = _EINTRA // _CL
_FULL_T = _NCHUNK // _NW
_EXTRA = _NCHUNK - _FULL_T * _NW

_BA_CHUNK = 8192
_BA_SLAB = _NN // _NW
_BA_FULL = _BA_SLAB // _BA_CHUNK
_BA_TAIL = _BA_SLAB - _BA_FULL * _BA_CHUNK


def _base_tables():
    edges, dists = [], []
    for i in range(_PL):
        for j in range(_PL):
            d = abs(i - j)
            if 0 < d <= _LW:
                edges.append((i, j))
                dists.append(d - 1)
    e = np.array(edges, np.int32).T
    return e[0], e[1], np.array(dists, np.int32)


_BI, _BJ, _BD = _base_tables()
_U = np.arange(4 * _E0)
_PAT4_NP = np.stack([
    16 * (_U // _E0) + _BI[_U % _E0],
    16 * (_U // _E0) + _BJ[_U % _E0],
]).astype(np.int32)
_DIST4_NP = _BD[_U % _E0].astype(np.int32)

_MESH = plsc.VectorSubcoreMesh(core_axis_name="c", subcore_axis_name="s")


@functools.partial(
    pl.kernel,
    mesh=_MESH,
    out_type=[
        jax.ShapeDtypeStruct((2 * _NN,), jnp.float32),
        jax.ShapeDtypeStruct((2, _ET), jnp.int32),
        jax.ShapeDtypeStruct((_ET,), jnp.float32),
        jax.ShapeDtypeStruct((_NN,), jnp.int32),
    ],
    scratch_types=[
        pltpu.VMEM((2 * _L,), jnp.float32),
        pltpu.VMEM((2 * _L,), jnp.float32),
        pltpu.VMEM((16352,), jnp.float32),
        pltpu.VMEM((16320,), jnp.float32),
        pltpu.VMEM((2, _CL), jnp.int32),
        pltpu.VMEM((_CL,), jnp.float32),
        pltpu.VMEM((_BA_CHUNK,), jnp.int32),
        pltpu.VMEM((2, 2048), jnp.int32),
        pltpu.VMEM((2048,), jnp.float32),
        pltpu.VMEM((16,), jnp.float32),
        pltpu.VMEM((2, 4 * _E0), jnp.int32),
        pltpu.VMEM((4 * _E0,), jnp.int32),
        pltpu.SemaphoreType.DMA,
        pltpu.SemaphoreType.DMA,
        pltpu.SemaphoreType.DMA,
    ],
)
def _sc_build(xt, wpad, pat, dist,
              nf, ei, attr, batch,
              xva, xvb, stga, stgb, ei_stg, at_stg, ba_stg,
              cei_stg, cat_stg, wv, patv, distv,
              at_sem, sema, semb):
    w = lax.axis_index("s") * 2 + lax.axis_index("c")
    iota = lax.iota(jnp.int32, 16)

    pltpu.sync_copy(wpad, wv)
    pltpu.sync_copy(pat, patv)
    pltpu.sync_copy(dist, distv)
    wvec = wv[...]

    for v in range(27):
        dvec = distv[pl.ds(16 * v, 16)]
        at_stg[pl.ds(16 * v, 16)] = wvec.at[dvec].get(
            mode="promise_in_bounds")
    sz = 432
    while sz < _CL:
        n = min(sz, _CL - sz)

        def _dup(i, _, sz=sz):
            at_stg[pl.ds(sz + 16 * i, 16)] = at_stg[pl.ds(16 * i, 16)]
            return 0

        lax.fori_loop(0, n // 16, _dup, 0)
        sz += n

    def _at_dma(t, _):
        cid = w + _NW * t
        pltpu.async_copy(at_stg, attr.at[pl.ds(_CL * cid, _CL)], at_sem)
        return 0

    lax.fori_loop(0, _FULL_T, _at_dma, 0)

    @pl.when(w < _EXTRA)
    def _():
        _at_dma(_FULL_T, 0)

    cw = wvec.at[jnp.full((16,), 4, jnp.int32)].get(
        mode="promise_in_bounds")

    def _cat_fill(v, _):
        cat_stg[pl.ds(16 * v, 16)] = cw
        return 0

    lax.fori_loop(0, 128, _cat_fill, 0)

    cb1 = 2040 * w + 1020
    cb2 = 2040 * w + 2040

    def _cb_fill(v, _):
        m = jnp.full((16,), 2048 * w + 16 * v, jnp.int32) + iota
        step1 = jnp.minimum(jnp.maximum(m - (cb1 - 1), 0), 1)
        step2 = jnp.minimum(jnp.maximum(m - (cb2 - 1), 0), 1)
        s = 16 * m + (32 * w + 15) + 16 * step1 + 16 * step2
        cei_stg[0, pl.ds(16 * v, 16)] = s
        cei_stg[1, pl.ds(16 * v, 16)] = s + 1
        return 0

    lax.fori_loop(0, 128, _cb_fill, 0)

    @pl.when(w < _NW - 1)
    def _():
        pltpu.sync_copy(cei_stg,
                        ei.at[:, pl.ds(_EINTRA + 2048 * w, 2048)])
        pltpu.sync_copy(cat_stg,
                        attr.at[pl.ds(_EINTRA + 2048 * w, 2048)])

    @pl.when(w == _NW - 1)
    def _():
        pltpu.sync_copy(
            cei_stg.at[:, pl.ds(0, _CROSS - 2048 * (_NW - 1))],
            ei.at[:, pl.ds(_EINTRA + 2048 * (_NW - 1),
                           _CROSS - 2048 * (_NW - 1))])
        pltpu.sync_copy(
            cat_stg.at[pl.ds(0, _CROSS - 2048 * (_NW - 1))],
            attr.at[pl.ds(_EINTRA + 2048 * (_NW - 1),
                          _CROSS - 2048 * (_NW - 1))])

    for c in range(_BA_FULL + 1):
        nvec = (_BA_CHUNK if c < _BA_FULL else _BA_TAIL) // 16
        g0 = (_BA_SLAB // 16) * w + (_BA_CHUNK // 16) * c

        def _baf(i, _, g0=g0):
            ba_stg[pl.ds(16 * i, 16)] = jnp.full((16,), g0 + i, jnp.int32)
            return 0

        lax.fori_loop(0, nvec, _baf, 0)
        pltpu.sync_copy(
            ba_stg.at[pl.ds(0, nvec * 16)],
            batch.at[pl.ds(_BA_SLAB * w + _BA_CHUNK * c, nvec * 16)])

    _ROW = 2 * _P * _PL
    _HA, _HB = 16352, 16320

    def _nf_fillA(q, _):
        sa = xva[pl.ds(16 * q, 16)]
        sb = xva[pl.ds(16 * q + 16, 16)]
        sc_ = xvb[pl.ds(16 * q, 16)]
        sd = xvb[pl.ds(16 * q + 16, 16)]
        stga[pl.ds(64 * q, 16)] = sa
        stga[pl.ds(64 * q + 16, 16)] = sb
        stga[pl.ds(64 * q + 32, 16)] = sc_
        stga[pl.ds(64 * q + 48, 16)] = sd
        return 0

    def _nf_fillB(q, _):
        sa = xvb[pl.ds(4080 + 16 * q, 16)]
        sb = xvb[pl.ds(4096 + 16 * q, 16)]
        sc_ = xva[pl.ds(4096 + 16 * q, 16)]
        sd = xva[pl.ds(4112 + 16 * q, 16)]
        stgb[pl.ds(64 * q, 16)] = sa
        stgb[pl.ds(64 * q + 16, 16)] = sb
        stgb[pl.ds(64 * q + 32, 16)] = sc_
        stgb[pl.ds(64 * q + 48, 16)] = sd
        return 0

    handles = []
    for j in range(2):
        b = 2 * w + j
        pltpu.sync_copy(xt.at[pl.ds(2 * _L * b, 2 * _L)], xva)
        pltpu.sync_copy(xt.at[pl.ds(2 * _L * b + 8, 2 * _L - 8)],
                        xvb.at[pl.ds(0, 2 * _L - 8)])
        if j == 1:
            handles[0].wait()
        lax.fori_loop(0, 255, _nf_fillA, 0)
        stga[pl.ds(16320, 16)] = xva[pl.ds(4080, 16)]
        stga[pl.ds(16336, 16)] = xva[pl.ds(4096, 16)]
        ha = pltpu.async_copy(stga, nf.at[pl.ds(_ROW * b, _HA)], sema)
        if j == 1:
            handles[1].wait()
        lax.fori_loop(0, 255, _nf_fillB, 0)
        hb = pltpu.async_copy(stgb, nf.at[pl.ds(_ROW * b + _HA, _HB)], semb)
        handles = [ha, hb]
    ha2, hb2 = handles

    pv = [[patv[r, pl.ds(16 * v, 16)] for v in range(27)] for r in range(2)]

    def _fill(q, g0):
        s = jnp.full((16,), 16 * (g0 + 4 * q), jnp.int32)
        for r in range(2):
            for v in range(27):
                ei_stg[r, pl.ds(432 * q + 16 * v, 16)] = pv[r][v] + s
        return g0

    def _eic(t, _):
        cid = w + _NW * t
        lax.fori_loop(0, 16, _fill, 64 * cid)
        pltpu.sync_copy(ei_stg, ei.at[:, pl.ds(_CL * cid, _CL)])
        return 0

    lax.fori_loop(0, _FULL_T, _eic, 0)

    @pl.when(w < _EXTRA)
    def _():
        _eic(_FULL_T, 0)

    ha2.wait()
    hb2.wait()

    def _at_drain(t, _):
        pltpu.make_async_copy(
            attr.at[pl.ds(0, _CL)], at_stg, at_sem).wait()
        return 0

    lax.fori_loop(0, _FULL_T, _at_drain, 0)

    @pl.when(w < _EXTRA)
    def _():
        _at_drain(_FULL_T, 0)


def kernel(iq_signal, edge_weights, cross_patch_weight):
    wpad = (jnp.zeros((16,), jnp.float32)
            .at[:4].set(edge_weights)
            .at[4].set(cross_patch_weight))
    xt = jnp.transpose(iq_signal, (0, 2, 1)).reshape(-1)
    nf_flat, ei, attr, batch = _sc_build(
        xt, wpad, jnp.asarray(_PAT4_NP), jnp.asarray(_DIST4_NP))
    return nf_flat.reshape(_NN, 2), ei, attr, batch
